# Initial kernel scaffold; baseline (speedup 1.0000x reference)
import jax
import jax.numpy as jnp
from jax.experimental import pallas as pl


def kernel(codes, table):
    raise NotImplementedError("write your pallas kernel here")


# speedup vs baseline: 1.8756x; 1.8756x over previous
import functools

import jax
import jax.numpy as jnp
from jax import lax
from jax.experimental import pallas as pl
from jax.experimental.pallas import tpu as pltpu
from jax.experimental.pallas import tpu_sc as plsc

NC = 2
NS = 16
NW = NC * NS

DIM = 256
CH = 128
NB = 2


def _make_lookup(B: int):
  assert B % (NW * CH) == 0
  bpw = B // NW
  iters = bpw // CH
  assert iters % NB == 0
  mesh = plsc.VectorSubcoreMesh(core_axis_name="c", subcore_axis_name="s")

  @functools.partial(
      pl.kernel,
      mesh=mesh,
      out_type=jax.ShapeDtypeStruct((NW, iters, CH, DIM), jnp.float32),
      scratch_types=[
          pltpu.VMEM((iters, CH), jnp.int32),
          pltpu.VMEM((NB, CH, DIM), jnp.float32),
          pltpu.SemaphoreType.DMA,
          pltpu.SemaphoreType.DMA,
          pltpu.SemaphoreType.DMA,
          pltpu.SemaphoreType.DMA,
      ],
  )
  def lookup(codes_hbm, table_hbm, out_hbm, idx_v, rows_v,
             gsem0, gsem1, osem0, osem1):
    gsems = (gsem0, gsem1)
    osems = (osem0, osem1)
    wid = lax.axis_index("s") * NC + lax.axis_index("c")

    pltpu.sync_copy(codes_hbm.at[wid], idx_v)

    for b in range(NB):
      pltpu.async_copy(table_hbm.at[idx_v.at[b]], rows_v.at[b], gsems[b])

    def group(go, carry):
      for b in range(NB):
        g = go * NB + b
        pltpu.make_async_copy(
            table_hbm.at[idx_v.at[g]], rows_v.at[b], gsems[b]).wait()
        pltpu.async_copy(rows_v.at[b], out_hbm.at[wid, g], osems[b])
        pltpu.make_async_copy(rows_v.at[b], out_hbm.at[wid, g],
                              osems[b]).wait()

        @pl.when(g + NB < iters)
        def _():
          pltpu.async_copy(
              table_hbm.at[idx_v.at[g + NB]], rows_v.at[b], gsems[b])

      return carry

    lax.fori_loop(0, iters // NB, group, 0)

  return lookup


def kernel(codes, table):
  batch, hist = codes.shape
  B = batch * hist
  codes_r = codes.reshape(NW, B // (NW * CH), CH).astype(jnp.int32)
  out = _make_lookup(B)(codes_r, table)
  return out.reshape(batch, hist, DIM)
